# baseline (speedup 1.0000x reference)
import jax
import jax.numpy as jnp
from jax import lax
from jax.experimental import pallas as pl
from jax.experimental.pallas import tpu as pltpu
from jax.experimental.pallas import tpu_sc as plsc

_NC = 2
_NS = 16
_NW = _NC * _NS

_C = 128
_G = 16

_SC_PARAMS = pltpu.CompilerParams(use_tc_tiling_on_sc=False)


def _scatter_pipeline(edges, wid, n_chunks, idx_bufs, do_chunk, drain):
  n_groups = n_chunks // _G

  def group(g, carry):
    for ref, plane in idx_bufs:
      pltpu.sync_copy(edges.at[plane, wid, pl.ds(g * _G, _G)], ref)
    do_chunk(g)
    drain()
    return carry

  lax.fori_loop(0, n_groups, group, 0)


def _seg_sum_sc(d_model, n_nodes, n_chunks, nbuf):
  rpt = n_nodes // _NS
  n_acc = n_nodes + 8
  mesh = plsc.VectorSubcoreMesh(core_axis_name="c", subcore_axis_name="s")

  scratch = [
      pltpu.VMEM_SHARED((n_acc, d_model), jnp.float32),
      pltpu.VMEM((_G, _C), jnp.int32),
      pltpu.VMEM((_G, _C), jnp.int32),
  ]
  scratch += [pltpu.VMEM((_C, d_model), jnp.float32) for _ in range(nbuf)]
  scratch += [pltpu.SemaphoreType.DMA for _ in range(2 * nbuf)]

  def body(y, edges, zrow, seg_out, acc, src_v, dst_v, *bufs):
    rows = bufs[:nbuf]
    gsem = bufs[nbuf:2 * nbuf]
    ssem = bufs[2 * nbuf:]
    cid = lax.axis_index("c")
    sid = lax.axis_index("s")
    wid = sid * _NC + cid

    pltpu.sync_copy(zrow, acc.at[pl.ds(sid * rpt, rpt)])
    plsc.subcore_barrier()

    gath = [None] * nbuf
    scat = [None] * nbuf

    def do_chunk(g):
      del g
      for q in range(min(nbuf - 1, _G)):
        gath[q] = pltpu.async_copy(y.at[src_v.at[q]], rows[q], gsem[q])
      for j in range(_G):
        p = j % nbuf
        k = j + nbuf - 1
        if k < _G:
          q = k % nbuf
          if scat[q] is not None:
            scat[q].wait()
            scat[q] = None
          gath[q] = pltpu.async_copy(y.at[src_v.at[k]], rows[q], gsem[q])
        gath[p].wait()
        scat[p] = pltpu.make_async_copy(rows[p], acc.at[dst_v.at[j]],
                                        ssem[p])
        scat[p].start(add=True)

    def drain():
      for q in range(nbuf):
        if scat[q] is not None:
          scat[q].wait()
          scat[q] = None

    _scatter_pipeline(edges, wid, n_chunks,
                      [(src_v, 0), (dst_v, 1)], do_chunk, drain)
    plsc.subcore_barrier()

    wb = (n_nodes // _NS) // 8 * 8
    tail = n_nodes - _NS * wb
    pltpu.sync_copy(acc.at[pl.ds(sid * wb, wb)],
                    seg_out.at[cid, pl.ds(sid * wb, wb)])
    if tail:
      @pl.when(sid == 0)
      def _():
        pltpu.sync_copy(acc.at[pl.ds(_NS * wb, tail)],
                        seg_out.at[cid, pl.ds(_NS * wb, tail)])

  return pl.kernel(
      body,
      out_type=[jax.ShapeDtypeStruct((_NC, n_nodes, d_model), jnp.float32)],
      mesh=mesh, scratch_types=scratch, compiler_params=_SC_PARAMS)


def _cnt_sc(n_nodes, n_chunks):
  rpt = n_nodes // _NS
  n_acc = n_nodes + 8
  mesh = plsc.VectorSubcoreMesh(core_axis_name="c", subcore_axis_name="s")

  scratch = [
      pltpu.VMEM_SHARED((n_acc, 16), jnp.float32),
      pltpu.VMEM((_G, _C), jnp.int32),
      pltpu.VMEM((_C, 16), jnp.float32),
      pltpu.SemaphoreType.DMA,
      pltpu.SemaphoreType.DMA,
  ]

  def body(edges, ones, zcnt, cnt_out, cnt_sh, dst_v, ones_v, s0, s1):
    sems = (s0, s1)
    cid = lax.axis_index("c")
    sid = lax.axis_index("s")
    wid = sid * _NC + cid

    pltpu.sync_copy(ones, ones_v)
    pltpu.sync_copy(zcnt, cnt_sh.at[pl.ds(sid * rpt, rpt)])
    plsc.subcore_barrier()

    scat = [None, None]

    def do_chunk(g):
      del g
      for j in range(_G):
        p = j % 2
        if scat[p] is not None:
          scat[p].wait()
        scat[p] = pltpu.make_async_copy(ones_v, cnt_sh.at[dst_v.at[j]],
                                        sems[p])
        scat[p].start(add=True)

    def drain():
      for p in range(2):
        if scat[p] is not None:
          scat[p].wait()
          scat[p] = None

    _scatter_pipeline(edges, wid, n_chunks, [(dst_v, 1)], do_chunk, drain)
    plsc.subcore_barrier()

    wb = (n_nodes // _NS) // 8 * 8
    tail = n_nodes - _NS * wb
    pltpu.sync_copy(cnt_sh.at[pl.ds(sid * wb, wb)],
                    cnt_out.at[cid, pl.ds(sid * wb, wb)])
    if tail:
      @pl.when(sid == 0)
      def _():
        pltpu.sync_copy(cnt_sh.at[pl.ds(_NS * wb, tail)],
                        cnt_out.at[cid, pl.ds(_NS * wb, tail)])

  return pl.kernel(
      body,
      out_type=[jax.ShapeDtypeStruct((_NC, n_nodes, 16), jnp.float32)],
      mesh=mesh, scratch_types=scratch, compiler_params=_SC_PARAMS)


def _proj(x, W, b, block_n):
  n, d_in = x.shape
  d_out = W.shape[1]

  def body(x_ref, w_ref, *rest):
    out_ref = rest[-1]
    r = jnp.dot(x_ref[...], w_ref[...], preferred_element_type=jnp.float32)
    if b is not None:
      r = r + rest[0][...]
    out_ref[...] = r

  in_specs = [
      pl.BlockSpec((block_n, d_in), lambda i: (i, 0)),
      pl.BlockSpec((d_in, d_out), lambda i: (0, 0)),
  ]
  args = [x, W]
  if b is not None:
    in_specs.append(pl.BlockSpec((1, d_out), lambda i: (0, 0)))
    args.append(b.reshape(1, -1))
  return pl.pallas_call(
      body,
      grid=(n // block_n,),
      in_specs=in_specs,
      out_specs=pl.BlockSpec((block_n, d_out), lambda i: (i, 0)),
      out_shape=jax.ShapeDtypeStruct((n, d_out), jnp.float32),
  )(*args)


def _mid_layer(seg, cnt, z1, W2l, W2r, b2, block_n):
  _, n, d_h = seg.shape
  d_out = W2l.shape[1]

  def body(s_ref, c_ref, z1_ref, wl_ref, wr_ref, b_ref, y2_ref, z2_ref):
    c = jnp.maximum(c_ref[0, :, :1] + c_ref[1, :, :1], 1.0)
    mean = (s_ref[0] + s_ref[1]) / c
    h = jnp.maximum(mean + z1_ref[...], 0.0)
    y2_ref[...] = jnp.dot(h, wl_ref[...], preferred_element_type=jnp.float32)
    z2_ref[...] = (jnp.dot(h, wr_ref[...], preferred_element_type=jnp.float32)
                   + b_ref[...])

  return pl.pallas_call(
      body,
      grid=(n // block_n,),
      in_specs=[
          pl.BlockSpec((2, block_n, d_h), lambda i: (0, i, 0)),
          pl.BlockSpec((2, block_n, 16), lambda i: (0, i, 0)),
          pl.BlockSpec((block_n, d_h), lambda i: (i, 0)),
          pl.BlockSpec((d_h, d_out), lambda i: (0, 0)),
          pl.BlockSpec((d_h, d_out), lambda i: (0, 0)),
          pl.BlockSpec((1, d_out), lambda i: (0, 0)),
      ],
      out_specs=[
          pl.BlockSpec((block_n, d_out), lambda i: (i, 0)),
          pl.BlockSpec((block_n, d_out), lambda i: (i, 0)),
      ],
      out_shape=[
          jax.ShapeDtypeStruct((n, d_out), jnp.float32),
          jax.ShapeDtypeStruct((n, d_out), jnp.float32),
      ],
  )(seg, cnt, z1, W2l, W2r, b2.reshape(1, -1))


def _final_layer(seg, cnt, z2, block_n):
  _, n, d_out = seg.shape

  def body(s_ref, c_ref, z2_ref, out_ref):
    c = jnp.maximum(c_ref[0, :, :1] + c_ref[1, :, :1], 1.0)
    v = (s_ref[0] + s_ref[1]) / c + z2_ref[...]
    m = jnp.max(v, axis=1, keepdims=True)
    e = jnp.exp(v - m)
    s = jnp.sum(e, axis=1, keepdims=True)
    out_ref[...] = v - m - jnp.log(s)

  return pl.pallas_call(
      body,
      grid=(n // block_n,),
      in_specs=[
          pl.BlockSpec((2, block_n, d_out), lambda i: (0, i, 0)),
          pl.BlockSpec((2, block_n, 16), lambda i: (0, i, 0)),
          pl.BlockSpec((block_n, d_out), lambda i: (i, 0)),
      ],
      out_specs=pl.BlockSpec((block_n, d_out), lambda i: (i, 0)),
      out_shape=jax.ShapeDtypeStruct((n, d_out), jnp.float32),
  )(seg, cnt, z2)


def kernel(x, edge_index, W1l, W1r, b1, W2l, W2r, b2):
  n, d_in = x.shape
  e = edge_index.shape[1]
  d_h = W1l.shape[1]
  d_out = W2l.shape[1]

  n_chunks = -(-e // (_NW * _C))
  n_chunks = -(-n_chunks // _G) * _G
  e_pad = _NW * n_chunks * _C
  block_n = 1000

  dummy = jnp.array([[0], [n]], jnp.int32)
  e4 = jnp.concatenate(
      [edge_index, jnp.broadcast_to(dummy, (2, e_pad - e))], axis=1
  ).reshape(2, _NW, n_chunks, _C)
  ones = jnp.ones((_C, 16), jnp.float32)
  zrow_h = jnp.zeros((n // _NS, d_h), jnp.float32)
  zrow_o = jnp.zeros((n // _NS, d_out), jnp.float32)
  zcnt = jnp.zeros((n // _NS, 16), jnp.float32)

  (cnt,) = _cnt_sc(n, n_chunks)(e4, ones, zcnt)

  y1 = _proj(x, W1l, None, block_n)
  (seg1,) = _seg_sum_sc(d_h, n, n_chunks, 2)(y1, e4, zrow_h)
  z1 = _proj(x, W1r, b1, block_n)
  y2, z2 = _mid_layer(seg1, cnt, z1, W2l, W2r, b2, block_n)

  (seg2,) = _seg_sum_sc(d_out, n, n_chunks, 4)(y2, e4, zrow_o)
  return _final_layer(seg2, cnt, z2, block_n)


# speedup vs baseline: 2.8374x; 2.8374x over previous
import jax
import jax.numpy as jnp
from jax import lax
from jax.experimental import pallas as pl
from jax.experimental.pallas import tpu as pltpu
from jax.experimental.pallas import tpu_sc as plsc

_NC = 2
_NS = 16
_NW = _NC * _NS

_C = 125
_G = 16

_SC_PARAMS = pltpu.CompilerParams(use_tc_tiling_on_sc=False)


def _scatter_pipeline(edges, wid, n_chunks, idx_bufs, do_chunk, drain):
  n_groups = n_chunks // _G

  def group(g, carry):
    for ref, plane in idx_bufs:
      pltpu.sync_copy(edges.at[plane, wid, pl.ds(g * _G, _G)], ref)
    do_chunk(g)
    drain()
    return carry

  lax.fori_loop(0, n_groups, group, 0)


def _seg_sum_sc(d_model, n_nodes, n_chunks, nbuf):
  rpt = n_nodes // _NS
  mesh = plsc.VectorSubcoreMesh(core_axis_name="c", subcore_axis_name="s")

  scratch = [
      pltpu.VMEM_SHARED((n_nodes, d_model), jnp.float32),
      pltpu.VMEM((_G, _C), jnp.int32),
      pltpu.VMEM((_G, _C), jnp.int32),
  ]
  scratch += [pltpu.VMEM((_C, d_model), jnp.float32) for _ in range(nbuf)]
  scratch += [pltpu.SemaphoreType.DMA for _ in range(2 * nbuf)]

  def body(y, edges, zrow, seg_out, acc, src_v, dst_v, *bufs):
    rows = bufs[:nbuf]
    gsem = bufs[nbuf:2 * nbuf]
    ssem = bufs[2 * nbuf:]
    cid = lax.axis_index("c")
    sid = lax.axis_index("s")
    wid = sid * _NC + cid

    pltpu.sync_copy(zrow, acc.at[pl.ds(sid * rpt, rpt)])
    plsc.subcore_barrier()

    gath = [None] * nbuf
    scat = [None] * nbuf

    def do_chunk(g):
      del g
      for q in range(min(nbuf - 1, _G)):
        gath[q] = pltpu.async_copy(y.at[src_v.at[q]], rows[q], gsem[q])
      for j in range(_G):
        p = j % nbuf
        k = j + nbuf - 1
        if k < _G:
          q = k % nbuf
          if scat[q] is not None:
            scat[q].wait()
            scat[q] = None
          gath[q] = pltpu.async_copy(y.at[src_v.at[k]], rows[q], gsem[q])
        gath[p].wait()
        scat[p] = pltpu.make_async_copy(rows[p], acc.at[dst_v.at[j]],
                                        ssem[p])
        scat[p].start(add=True)

    def drain():
      for q in range(nbuf):
        if scat[q] is not None:
          scat[q].wait()
          scat[q] = None

    _scatter_pipeline(edges, wid, n_chunks,
                      [(src_v, 0), (dst_v, 1)], do_chunk, drain)
    plsc.subcore_barrier()

    wb = (n_nodes // _NS) // 8 * 8
    tail = n_nodes - _NS * wb
    pltpu.sync_copy(acc.at[pl.ds(sid * wb, wb)],
                    seg_out.at[cid, pl.ds(sid * wb, wb)])
    if tail:
      @pl.when(sid == 0)
      def _():
        pltpu.sync_copy(acc.at[pl.ds(_NS * wb, tail)],
                        seg_out.at[cid, pl.ds(_NS * wb, tail)])

  return pl.kernel(
      body,
      out_type=[jax.ShapeDtypeStruct((_NC, n_nodes, d_model), jnp.float32)],
      mesh=mesh, scratch_types=scratch, compiler_params=_SC_PARAMS)


def _cnt_sc(n_nodes, n_chunks):
  rpt = n_nodes // _NS
  mesh = plsc.VectorSubcoreMesh(core_axis_name="c", subcore_axis_name="s")

  scratch = [
      pltpu.VMEM_SHARED((n_nodes, 16), jnp.float32),
      pltpu.VMEM((_G, _C), jnp.int32),
      pltpu.VMEM((_C, 16), jnp.float32),
      pltpu.SemaphoreType.DMA,
      pltpu.SemaphoreType.DMA,
  ]

  def body(edges, ones, zcnt, cnt_out, cnt_sh, dst_v, ones_v, s0, s1):
    sems = (s0, s1)
    cid = lax.axis_index("c")
    sid = lax.axis_index("s")
    wid = sid * _NC + cid

    pltpu.sync_copy(ones, ones_v)
    pltpu.sync_copy(zcnt, cnt_sh.at[pl.ds(sid * rpt, rpt)])
    plsc.subcore_barrier()

    scat = [None, None]

    def do_chunk(g):
      del g
      for j in range(_G):
        p = j % 2
        if scat[p] is not None:
          scat[p].wait()
        scat[p] = pltpu.make_async_copy(ones_v, cnt_sh.at[dst_v.at[j]],
                                        sems[p])
        scat[p].start(add=True)

    def drain():
      for p in range(2):
        if scat[p] is not None:
          scat[p].wait()
          scat[p] = None

    _scatter_pipeline(edges, wid, n_chunks, [(dst_v, 1)], do_chunk, drain)
    plsc.subcore_barrier()

    wb = (n_nodes // _NS) // 8 * 8
    tail = n_nodes - _NS * wb
    pltpu.sync_copy(cnt_sh.at[pl.ds(sid * wb, wb)],
                    cnt_out.at[cid, pl.ds(sid * wb, wb)])
    if tail:
      @pl.when(sid == 0)
      def _():
        pltpu.sync_copy(cnt_sh.at[pl.ds(_NS * wb, tail)],
                        cnt_out.at[cid, pl.ds(_NS * wb, tail)])

  return pl.kernel(
      body,
      out_type=[jax.ShapeDtypeStruct((_NC, n_nodes, 16), jnp.float32)],
      mesh=mesh, scratch_types=scratch, compiler_params=_SC_PARAMS)


def _proj(x, W, b, block_n):
  n, d_in = x.shape
  d_out = W.shape[1]

  def body(x_ref, w_ref, *rest):
    out_ref = rest[-1]
    r = jnp.dot(x_ref[...], w_ref[...], preferred_element_type=jnp.float32)
    if b is not None:
      r = r + rest[0][...]
    out_ref[...] = r

  in_specs = [
      pl.BlockSpec((block_n, d_in), lambda i: (i, 0)),
      pl.BlockSpec((d_in, d_out), lambda i: (0, 0)),
  ]
  args = [x, W]
  if b is not None:
    in_specs.append(pl.BlockSpec((1, d_out), lambda i: (0, 0)))
    args.append(b.reshape(1, -1))
  return pl.pallas_call(
      body,
      grid=(n // block_n,),
      in_specs=in_specs,
      out_specs=pl.BlockSpec((block_n, d_out), lambda i: (i, 0)),
      out_shape=jax.ShapeDtypeStruct((n, d_out), jnp.float32),
  )(*args)


def _mid_layer(seg, cnt, z1, W2l, W2r, b2, block_n):
  _, n, d_h = seg.shape
  d_out = W2l.shape[1]

  def body(s_ref, c_ref, z1_ref, wl_ref, wr_ref, b_ref, y2_ref, z2_ref):
    c = jnp.maximum(c_ref[0, :, :1] + c_ref[1, :, :1], 1.0)
    mean = (s_ref[0] + s_ref[1]) / c
    h = jnp.maximum(mean + z1_ref[...], 0.0)
    y2_ref[...] = jnp.dot(h, wl_ref[...], preferred_element_type=jnp.float32)
    z2_ref[...] = (jnp.dot(h, wr_ref[...], preferred_element_type=jnp.float32)
                   + b_ref[...])

  return pl.pallas_call(
      body,
      grid=(n // block_n,),
      in_specs=[
          pl.BlockSpec((2, block_n, d_h), lambda i: (0, i, 0)),
          pl.BlockSpec((2, block_n, 16), lambda i: (0, i, 0)),
          pl.BlockSpec((block_n, d_h), lambda i: (i, 0)),
          pl.BlockSpec((d_h, d_out), lambda i: (0, 0)),
          pl.BlockSpec((d_h, d_out), lambda i: (0, 0)),
          pl.BlockSpec((1, d_out), lambda i: (0, 0)),
      ],
      out_specs=[
          pl.BlockSpec((block_n, d_out), lambda i: (i, 0)),
          pl.BlockSpec((block_n, d_out), lambda i: (i, 0)),
      ],
      out_shape=[
          jax.ShapeDtypeStruct((n, d_out), jnp.float32),
          jax.ShapeDtypeStruct((n, d_out), jnp.float32),
      ],
  )(seg, cnt, z1, W2l, W2r, b2.reshape(1, -1))


def _final_layer(seg, cnt, z2, block_n):
  _, n, d_out = seg.shape

  def body(s_ref, c_ref, z2_ref, out_ref):
    c = jnp.maximum(c_ref[0, :, :1] + c_ref[1, :, :1], 1.0)
    v = (s_ref[0] + s_ref[1]) / c + z2_ref[...]
    m = jnp.max(v, axis=1, keepdims=True)
    e = jnp.exp(v - m)
    s = jnp.sum(e, axis=1, keepdims=True)
    out_ref[...] = v - m - jnp.log(s)

  return pl.pallas_call(
      body,
      grid=(n // block_n,),
      in_specs=[
          pl.BlockSpec((2, block_n, d_out), lambda i: (0, i, 0)),
          pl.BlockSpec((2, block_n, 16), lambda i: (0, i, 0)),
          pl.BlockSpec((block_n, d_out), lambda i: (i, 0)),
      ],
      out_specs=pl.BlockSpec((block_n, d_out), lambda i: (i, 0)),
      out_shape=jax.ShapeDtypeStruct((n, d_out), jnp.float32),
  )(seg, cnt, z2)


def kernel(x, edge_index, W1l, W1r, b1, W2l, W2r, b2):
  n, d_in = x.shape
  e = edge_index.shape[1]
  d_h = W1l.shape[1]
  d_out = W2l.shape[1]

  n_chunks = e // (_NW * _C)
  block_n = 1000

  e4 = edge_index.reshape(2, _NW, n_chunks, _C)
  ones = jnp.ones((_C, 16), jnp.float32)
  zrow_h = jnp.zeros((n // _NS, d_h), jnp.float32)
  zrow_o = jnp.zeros((n // _NS, d_out), jnp.float32)
  zcnt = jnp.zeros((n // _NS, 16), jnp.float32)

  (cnt,) = _cnt_sc(n, n_chunks)(e4, ones, zcnt)

  y1 = _proj(x, W1l, None, block_n)
  (seg1,) = _seg_sum_sc(d_h, n, n_chunks, 2)(y1, e4, zrow_h)
  z1 = _proj(x, W1r, b1, block_n)
  y2, z2 = _mid_layer(seg1, cnt, z1, W2l, W2r, b2, block_n)

  (seg2,) = _seg_sum_sc(d_out, n, n_chunks, 4)(y2, e4, zrow_o)
  return _final_layer(seg2, cnt, z2, block_n)


# speedup vs baseline: 2.8618x; 1.0086x over previous
import jax
import jax.numpy as jnp
from jax import lax
from jax.experimental import pallas as pl
from jax.experimental.pallas import tpu as pltpu
from jax.experimental.pallas import tpu_sc as plsc

_NC = 2
_NS = 16
_NW = _NC * _NS

_C = 125
_G = 16

_SC_PARAMS = pltpu.CompilerParams(use_tc_tiling_on_sc=False)


def _scatter_pipeline(edges, wid, n_chunks, idx_bufs, do_chunk, drain):
  n_groups = n_chunks // _G

  def group(g, carry):
    for ref, plane in idx_bufs:
      pltpu.sync_copy(edges.at[plane, wid, pl.ds(g * _G, _G)], ref)
    do_chunk(g)
    drain()
    return carry

  lax.fori_loop(0, n_groups, group, 0)


def _seg_sum_sc(d_model, n_nodes, n_chunks, nbuf):
  rpt = n_nodes // _NS
  mesh = plsc.VectorSubcoreMesh(core_axis_name="c", subcore_axis_name="s")

  scratch = [
      pltpu.VMEM_SHARED((n_nodes, d_model), jnp.float32),
      pltpu.VMEM((_G, _C), jnp.int32),
      pltpu.VMEM((_G, _C), jnp.int32),
  ]
  scratch += [pltpu.VMEM((_C, d_model), jnp.float32) for _ in range(nbuf)]
  scratch += [pltpu.SemaphoreType.DMA for _ in range(2 * nbuf)]

  def body(y, edges, zrow, seg_out, acc, src_v, dst_v, *bufs):
    rows = bufs[:nbuf]
    gsem = bufs[nbuf:2 * nbuf]
    ssem = bufs[2 * nbuf:]
    cid = lax.axis_index("c")
    sid = lax.axis_index("s")
    wid = sid * _NC + cid

    pltpu.sync_copy(zrow, acc.at[pl.ds(sid * rpt, rpt)])
    plsc.subcore_barrier()

    gath = [None] * nbuf
    scat = [None] * nbuf

    def do_chunk(g):
      del g
      for q in range(min(nbuf - 1, _G)):
        gath[q] = pltpu.async_copy(y.at[src_v.at[q]], rows[q], gsem[q])
      for j in range(_G):
        p = j % nbuf
        k = j + nbuf - 1
        if k < _G:
          q = k % nbuf
          if scat[q] is not None:
            scat[q].wait()
            scat[q] = None
          gath[q] = pltpu.async_copy(y.at[src_v.at[k]], rows[q], gsem[q])
        gath[p].wait()
        scat[p] = pltpu.make_async_copy(rows[p], acc.at[dst_v.at[j]],
                                        ssem[p])
        scat[p].start(add=True)

    def drain():
      for q in range(nbuf):
        if scat[q] is not None:
          scat[q].wait()
          scat[q] = None

    _scatter_pipeline(edges, wid, n_chunks,
                      [(src_v, 0), (dst_v, 1)], do_chunk, drain)
    plsc.subcore_barrier()

    wb = (n_nodes // _NS) // 8 * 8
    tail = n_nodes - _NS * wb
    pltpu.sync_copy(acc.at[pl.ds(sid * wb, wb)],
                    seg_out.at[cid, pl.ds(sid * wb, wb)])
    if tail:
      @pl.when(sid == 0)
      def _():
        pltpu.sync_copy(acc.at[pl.ds(_NS * wb, tail)],
                        seg_out.at[cid, pl.ds(_NS * wb, tail)])

  return pl.kernel(
      body,
      out_type=[jax.ShapeDtypeStruct((_NC, n_nodes, d_model), jnp.float32)],
      mesh=mesh, scratch_types=scratch, compiler_params=_SC_PARAMS)


def _cnt_sc(n_nodes, n_chunks):
  rpt = n_nodes // _NS
  mesh = plsc.VectorSubcoreMesh(core_axis_name="c", subcore_axis_name="s")

  scratch = [
      pltpu.VMEM_SHARED((n_nodes, 16), jnp.float32),
      pltpu.VMEM((_G, _C), jnp.int32),
      pltpu.VMEM((_C, 16), jnp.float32),
      pltpu.SemaphoreType.DMA,
      pltpu.SemaphoreType.DMA,
  ]

  def body(edges, ones, zcnt, cnt_out, cnt_sh, dst_v, ones_v, s0, s1):
    sems = (s0, s1)
    cid = lax.axis_index("c")
    sid = lax.axis_index("s")
    wid = sid * _NC + cid

    pltpu.sync_copy(ones, ones_v)
    pltpu.sync_copy(zcnt, cnt_sh.at[pl.ds(sid * rpt, rpt)])
    plsc.subcore_barrier()

    scat = [None, None]

    def do_chunk(g):
      del g
      for j in range(_G):
        p = j % 2
        if scat[p] is not None:
          scat[p].wait()
        scat[p] = pltpu.make_async_copy(ones_v, cnt_sh.at[dst_v.at[j]],
                                        sems[p])
        scat[p].start(add=True)

    def drain():
      for p in range(2):
        if scat[p] is not None:
          scat[p].wait()
          scat[p] = None

    _scatter_pipeline(edges, wid, n_chunks, [(dst_v, 1)], do_chunk, drain)
    plsc.subcore_barrier()

    wb = (n_nodes // _NS) // 8 * 8
    tail = n_nodes - _NS * wb
    pltpu.sync_copy(cnt_sh.at[pl.ds(sid * wb, wb)],
                    cnt_out.at[cid, pl.ds(sid * wb, wb)])
    if tail:
      @pl.when(sid == 0)
      def _():
        pltpu.sync_copy(cnt_sh.at[pl.ds(_NS * wb, tail)],
                        cnt_out.at[cid, pl.ds(_NS * wb, tail)])

  return pl.kernel(
      body,
      out_type=[jax.ShapeDtypeStruct((_NC, n_nodes, 16), jnp.float32)],
      mesh=mesh, scratch_types=scratch, compiler_params=_SC_PARAMS)


def _proj(x, W, b, block_n):
  n, d_in = x.shape
  d_out = W.shape[1]

  def body(x_ref, w_ref, *rest):
    out_ref = rest[-1]
    r = jnp.dot(x_ref[...], w_ref[...], preferred_element_type=jnp.float32)
    if b is not None:
      r = r + rest[0][...]
    out_ref[...] = r

  in_specs = [
      pl.BlockSpec((block_n, d_in), lambda i: (i, 0)),
      pl.BlockSpec((d_in, d_out), lambda i: (0, 0)),
  ]
  args = [x, W]
  if b is not None:
    in_specs.append(pl.BlockSpec((1, d_out), lambda i: (0, 0)))
    args.append(b.reshape(1, -1))
  return pl.pallas_call(
      body,
      grid=(n // block_n,),
      in_specs=in_specs,
      out_specs=pl.BlockSpec((block_n, d_out), lambda i: (i, 0)),
      out_shape=jax.ShapeDtypeStruct((n, d_out), jnp.float32),
  )(*args)


def _mid_layer(seg, cnt, z1, W1l, W2l, W2r, b2, block_n):
  _, n, d_in = seg.shape
  d_h = W1l.shape[1]
  d_out = W2l.shape[1]

  def body(s_ref, c_ref, z1_ref, w1_ref, wl_ref, wr_ref, b_ref,
           y2_ref, z2_ref):
    c = jnp.maximum(c_ref[0, :, :1] + c_ref[1, :, :1], 1.0)
    mean = (s_ref[0] + s_ref[1]) / c
    h = jnp.maximum(
        jnp.dot(mean, w1_ref[...], preferred_element_type=jnp.float32)
        + z1_ref[...], 0.0)
    y2_ref[...] = jnp.dot(h, wl_ref[...], preferred_element_type=jnp.float32)
    z2_ref[...] = (jnp.dot(h, wr_ref[...], preferred_element_type=jnp.float32)
                   + b_ref[...])

  return pl.pallas_call(
      body,
      grid=(n // block_n,),
      in_specs=[
          pl.BlockSpec((2, block_n, d_in), lambda i: (0, i, 0)),
          pl.BlockSpec((2, block_n, 16), lambda i: (0, i, 0)),
          pl.BlockSpec((block_n, d_h), lambda i: (i, 0)),
          pl.BlockSpec((d_in, d_h), lambda i: (0, 0)),
          pl.BlockSpec((d_h, d_out), lambda i: (0, 0)),
          pl.BlockSpec((d_h, d_out), lambda i: (0, 0)),
          pl.BlockSpec((1, d_out), lambda i: (0, 0)),
      ],
      out_specs=[
          pl.BlockSpec((block_n, d_out), lambda i: (i, 0)),
          pl.BlockSpec((block_n, d_out), lambda i: (i, 0)),
      ],
      out_shape=[
          jax.ShapeDtypeStruct((n, d_out), jnp.float32),
          jax.ShapeDtypeStruct((n, d_out), jnp.float32),
      ],
  )(seg, cnt, z1, W1l, W2l, W2r, b2.reshape(1, -1))


def _final_layer(seg, cnt, z2, block_n):
  _, n, d_out = seg.shape

  def body(s_ref, c_ref, z2_ref, out_ref):
    c = jnp.maximum(c_ref[0, :, :1] + c_ref[1, :, :1], 1.0)
    v = (s_ref[0] + s_ref[1]) / c + z2_ref[...]
    m = jnp.max(v, axis=1, keepdims=True)
    e = jnp.exp(v - m)
    s = jnp.sum(e, axis=1, keepdims=True)
    out_ref[...] = v - m - jnp.log(s)

  return pl.pallas_call(
      body,
      grid=(n // block_n,),
      in_specs=[
          pl.BlockSpec((2, block_n, d_out), lambda i: (0, i, 0)),
          pl.BlockSpec((2, block_n, 16), lambda i: (0, i, 0)),
          pl.BlockSpec((block_n, d_out), lambda i: (i, 0)),
      ],
      out_specs=pl.BlockSpec((block_n, d_out), lambda i: (i, 0)),
      out_shape=jax.ShapeDtypeStruct((n, d_out), jnp.float32),
  )(seg, cnt, z2)


def kernel(x, edge_index, W1l, W1r, b1, W2l, W2r, b2):
  n, d_in = x.shape
  e = edge_index.shape[1]
  d_h = W1l.shape[1]
  d_out = W2l.shape[1]

  n_chunks = e // (_NW * _C)
  block_n = 1000

  e4 = edge_index.reshape(2, _NW, n_chunks, _C)
  ones = jnp.ones((_C, 16), jnp.float32)
  zrow_h = jnp.zeros((n // _NS, d_in), jnp.float32)
  zrow_o = jnp.zeros((n // _NS, d_out), jnp.float32)
  zcnt = jnp.zeros((n // _NS, 16), jnp.float32)

  (cnt,) = _cnt_sc(n, n_chunks)(e4, ones, zcnt)

  (seg1,) = _seg_sum_sc(d_in, n, n_chunks, 2)(x, e4, zrow_h)
  z1 = _proj(x, W1r, b1, block_n)
  y2, z2 = _mid_layer(seg1, cnt, z1, W1l, W2l, W2r, b2, block_n)

  (seg2,) = _seg_sum_sc(d_out, n, n_chunks, 4)(y2, e4, zrow_o)
  return _final_layer(seg2, cnt, z2, block_n)


# speedup vs baseline: 2.8978x; 1.0126x over previous
import jax
import jax.numpy as jnp
from jax import lax
from jax.experimental import pallas as pl
from jax.experimental.pallas import tpu as pltpu
from jax.experimental.pallas import tpu_sc as plsc

_NC = 2
_NS = 16
_NW = _NC * _NS

_C = 100
_G = 20

_SC_PARAMS = pltpu.CompilerParams(use_tc_tiling_on_sc=False)


def _scatter_pipeline(edges, wid, n_chunks, idx_bufs, do_chunk, drain):
  n_groups = n_chunks // _G

  def group(g, carry):
    for ref, plane in idx_bufs:
      pltpu.sync_copy(edges.at[plane, wid, pl.ds(g * _G, _G)], ref)
    do_chunk(g)
    drain()
    return carry

  lax.fori_loop(0, n_groups, group, 0)


def _seg_sum_sc(d_model, n_nodes, n_chunks, nbuf):
  rpt = n_nodes // _NS
  mesh = plsc.VectorSubcoreMesh(core_axis_name="c", subcore_axis_name="s")

  scratch = [
      pltpu.VMEM_SHARED((n_nodes, d_model), jnp.float32),
      pltpu.VMEM((_G, _C), jnp.int32),
      pltpu.VMEM((_G, _C), jnp.int32),
  ]
  scratch += [pltpu.VMEM((_C, d_model), jnp.float32) for _ in range(nbuf)]
  scratch += [pltpu.SemaphoreType.DMA for _ in range(2 * nbuf)]

  def body(y, edges, zrow, seg_out, acc, src_v, dst_v, *bufs):
    rows = bufs[:nbuf]
    gsem = bufs[nbuf:2 * nbuf]
    ssem = bufs[2 * nbuf:]
    cid = lax.axis_index("c")
    sid = lax.axis_index("s")
    wid = sid * _NC + cid

    pltpu.sync_copy(zrow, acc.at[pl.ds(sid * rpt, rpt)])
    plsc.subcore_barrier()

    gath = [None] * nbuf
    scat = [None] * nbuf

    def do_chunk(g):
      del g
      for q in range(min(nbuf - 1, _G)):
        gath[q] = pltpu.async_copy(y.at[src_v.at[q]], rows[q], gsem[q])
      for j in range(_G):
        p = j % nbuf
        k = j + nbuf - 1
        if k < _G:
          q = k % nbuf
          if scat[q] is not None:
            scat[q].wait()
            scat[q] = None
          gath[q] = pltpu.async_copy(y.at[src_v.at[k]], rows[q], gsem[q])
        gath[p].wait()
        scat[p] = pltpu.make_async_copy(rows[p], acc.at[dst_v.at[j]],
                                        ssem[p])
        scat[p].start(add=True)

    def drain():
      for q in range(nbuf):
        if scat[q] is not None:
          scat[q].wait()
          scat[q] = None

    _scatter_pipeline(edges, wid, n_chunks,
                      [(src_v, 0), (dst_v, 1)], do_chunk, drain)
    plsc.subcore_barrier()

    wb = (n_nodes // _NS) // 8 * 8
    tail = n_nodes - _NS * wb
    pltpu.sync_copy(acc.at[pl.ds(sid * wb, wb)],
                    seg_out.at[cid, pl.ds(sid * wb, wb)])
    if tail:
      @pl.when(sid == 0)
      def _():
        pltpu.sync_copy(acc.at[pl.ds(_NS * wb, tail)],
                        seg_out.at[cid, pl.ds(_NS * wb, tail)])

  return pl.kernel(
      body,
      out_type=[jax.ShapeDtypeStruct((_NC, n_nodes, d_model), jnp.float32)],
      mesh=mesh, scratch_types=scratch, compiler_params=_SC_PARAMS)


def _cnt_sc(n_nodes, n_chunks):
  rpt = n_nodes // _NS
  mesh = plsc.VectorSubcoreMesh(core_axis_name="c", subcore_axis_name="s")

  scratch = [
      pltpu.VMEM_SHARED((n_nodes, 16), jnp.float32),
      pltpu.VMEM((_G, _C), jnp.int32),
      pltpu.VMEM((_C, 16), jnp.float32),
      pltpu.SemaphoreType.DMA,
      pltpu.SemaphoreType.DMA,
  ]

  def body(edges, ones, zcnt, cnt_out, cnt_sh, dst_v, ones_v, s0, s1):
    sems = (s0, s1)
    cid = lax.axis_index("c")
    sid = lax.axis_index("s")
    wid = sid * _NC + cid

    pltpu.sync_copy(ones, ones_v)
    pltpu.sync_copy(zcnt, cnt_sh.at[pl.ds(sid * rpt, rpt)])
    plsc.subcore_barrier()

    scat = [None, None]

    def do_chunk(g):
      del g
      for j in range(_G):
        p = j % 2
        if scat[p] is not None:
          scat[p].wait()
        scat[p] = pltpu.make_async_copy(ones_v, cnt_sh.at[dst_v.at[j]],
                                        sems[p])
        scat[p].start(add=True)

    def drain():
      for p in range(2):
        if scat[p] is not None:
          scat[p].wait()
          scat[p] = None

    _scatter_pipeline(edges, wid, n_chunks, [(dst_v, 1)], do_chunk, drain)
    plsc.subcore_barrier()

    wb = (n_nodes // _NS) // 8 * 8
    tail = n_nodes - _NS * wb
    pltpu.sync_copy(cnt_sh.at[pl.ds(sid * wb, wb)],
                    cnt_out.at[cid, pl.ds(sid * wb, wb)])
    if tail:
      @pl.when(sid == 0)
      def _():
        pltpu.sync_copy(cnt_sh.at[pl.ds(_NS * wb, tail)],
                        cnt_out.at[cid, pl.ds(_NS * wb, tail)])

  return pl.kernel(
      body,
      out_type=[jax.ShapeDtypeStruct((_NC, n_nodes, 16), jnp.float32)],
      mesh=mesh, scratch_types=scratch, compiler_params=_SC_PARAMS)


def _proj(x, W, b, block_n):
  n, d_in = x.shape
  d_out = W.shape[1]

  def body(x_ref, w_ref, *rest):
    out_ref = rest[-1]
    r = jnp.dot(x_ref[...], w_ref[...], preferred_element_type=jnp.float32)
    if b is not None:
      r = r + rest[0][...]
    out_ref[...] = r

  in_specs = [
      pl.BlockSpec((block_n, d_in), lambda i: (i, 0)),
      pl.BlockSpec((d_in, d_out), lambda i: (0, 0)),
  ]
  args = [x, W]
  if b is not None:
    in_specs.append(pl.BlockSpec((1, d_out), lambda i: (0, 0)))
    args.append(b.reshape(1, -1))
  return pl.pallas_call(
      body,
      grid=(n // block_n,),
      in_specs=in_specs,
      out_specs=pl.BlockSpec((block_n, d_out), lambda i: (i, 0)),
      out_shape=jax.ShapeDtypeStruct((n, d_out), jnp.float32),
  )(*args)


def _mid_layer(seg, cnt, z1, W1l, W2l, W2r, b2, block_n):
  _, n, d_in = seg.shape
  d_h = W1l.shape[1]
  d_out = W2l.shape[1]

  def body(s_ref, c_ref, z1_ref, w1_ref, wl_ref, wr_ref, b_ref,
           y2_ref, z2_ref):
    c = jnp.maximum(c_ref[0, :, :1] + c_ref[1, :, :1], 1.0)
    mean = (s_ref[0] + s_ref[1]) / c
    h = jnp.maximum(
        jnp.dot(mean, w1_ref[...], preferred_element_type=jnp.float32)
        + z1_ref[...], 0.0)
    y2_ref[...] = jnp.dot(h, wl_ref[...], preferred_element_type=jnp.float32)
    z2_ref[...] = (jnp.dot(h, wr_ref[...], preferred_element_type=jnp.float32)
                   + b_ref[...])

  return pl.pallas_call(
      body,
      grid=(n // block_n,),
      in_specs=[
          pl.BlockSpec((2, block_n, d_in), lambda i: (0, i, 0)),
          pl.BlockSpec((2, block_n, 16), lambda i: (0, i, 0)),
          pl.BlockSpec((block_n, d_h), lambda i: (i, 0)),
          pl.BlockSpec((d_in, d_h), lambda i: (0, 0)),
          pl.BlockSpec((d_h, d_out), lambda i: (0, 0)),
          pl.BlockSpec((d_h, d_out), lambda i: (0, 0)),
          pl.BlockSpec((1, d_out), lambda i: (0, 0)),
      ],
      out_specs=[
          pl.BlockSpec((block_n, d_out), lambda i: (i, 0)),
          pl.BlockSpec((block_n, d_out), lambda i: (i, 0)),
      ],
      out_shape=[
          jax.ShapeDtypeStruct((n, d_out), jnp.float32),
          jax.ShapeDtypeStruct((n, d_out), jnp.float32),
      ],
  )(seg, cnt, z1, W1l, W2l, W2r, b2.reshape(1, -1))


def _final_layer(seg, cnt, z2, block_n):
  _, n, d_out = seg.shape

  def body(s_ref, c_ref, z2_ref, out_ref):
    c = jnp.maximum(c_ref[0, :, :1] + c_ref[1, :, :1], 1.0)
    v = (s_ref[0] + s_ref[1]) / c + z2_ref[...]
    m = jnp.max(v, axis=1, keepdims=True)
    e = jnp.exp(v - m)
    s = jnp.sum(e, axis=1, keepdims=True)
    out_ref[...] = v - m - jnp.log(s)

  return pl.pallas_call(
      body,
      grid=(n // block_n,),
      in_specs=[
          pl.BlockSpec((2, block_n, d_out), lambda i: (0, i, 0)),
          pl.BlockSpec((2, block_n, 16), lambda i: (0, i, 0)),
          pl.BlockSpec((block_n, d_out), lambda i: (i, 0)),
      ],
      out_specs=pl.BlockSpec((block_n, d_out), lambda i: (i, 0)),
      out_shape=jax.ShapeDtypeStruct((n, d_out), jnp.float32),
  )(seg, cnt, z2)


def kernel(x, edge_index, W1l, W1r, b1, W2l, W2r, b2):
  n, d_in = x.shape
  e = edge_index.shape[1]
  d_h = W1l.shape[1]
  d_out = W2l.shape[1]

  n_chunks = e // (_NW * _C)
  block_n = 1000

  e4 = edge_index.reshape(2, _NW, n_chunks, _C)
  ones = jnp.ones((_C, 16), jnp.float32)
  zrow_h = jnp.zeros((n // _NS, d_in), jnp.float32)
  zrow_o = jnp.zeros((n // _NS, d_out), jnp.float32)
  zcnt = jnp.zeros((n // _NS, 16), jnp.float32)

  (cnt,) = _cnt_sc(n, n_chunks)(e4, ones, zcnt)

  (seg1,) = _seg_sum_sc(d_in, n, n_chunks, 3)(x, e4, zrow_h)
  z1 = _proj(x, W1r, b1, block_n)
  y2, z2 = _mid_layer(seg1, cnt, z1, W1l, W2l, W2r, b2, block_n)

  (seg2,) = _seg_sum_sc(d_out, n, n_chunks, 6)(y2, e4, zrow_o)
  return _final_layer(seg2, cnt, z2, block_n)


# speedup vs baseline: 3.1427x; 1.0845x over previous
import jax
import jax.numpy as jnp
from jax import lax
from jax.experimental import pallas as pl
from jax.experimental.pallas import tpu as pltpu
from jax.experimental.pallas import tpu_sc as plsc

_NC = 2
_NS = 16
_NW = _NC * _NS

_C = 100
_G = 20

_SC_PARAMS = pltpu.CompilerParams(use_tc_tiling_on_sc=False)


def _scatter_pipeline(edges, wid, n_chunks, idx_bufs, do_chunk, drain):
  n_groups = n_chunks // _G

  def group(g, carry):
    for ref, plane in idx_bufs:
      pltpu.sync_copy(edges.at[plane, wid, pl.ds(g * _G, _G)], ref)
    do_chunk(g)
    drain()
    return carry

  lax.fori_loop(0, n_groups, group, 0)


def _seg_sum_sc(d_model, n_nodes, n_chunks, nbuf):
  rpt = n_nodes // _NS
  mesh = plsc.VectorSubcoreMesh(core_axis_name="c", subcore_axis_name="s")

  scratch = [
      pltpu.VMEM_SHARED((n_nodes, d_model), jnp.float32),
      pltpu.VMEM((_G, _C), jnp.int32),
      pltpu.VMEM((_G, _C), jnp.int32),
  ]
  scratch += [pltpu.VMEM((_C, d_model), jnp.float32) for _ in range(nbuf)]
  scratch += [pltpu.SemaphoreType.DMA for _ in range(2 * nbuf)]

  def body(y, edges, zrow, seg_out, acc, src_v, dst_v, *bufs):
    rows = bufs[:nbuf]
    gsem = bufs[nbuf:2 * nbuf]
    ssem = bufs[2 * nbuf:]
    cid = lax.axis_index("c")
    sid = lax.axis_index("s")
    wid = sid * _NC + cid

    pltpu.sync_copy(zrow, acc.at[pl.ds(sid * rpt, rpt)])
    plsc.subcore_barrier()

    gath = [None] * nbuf
    scat = [None] * nbuf

    def do_chunk(g):
      del g
      for q in range(min(nbuf - 1, _G)):
        gath[q] = pltpu.async_copy(y.at[src_v.at[q]], rows[q], gsem[q])
      for j in range(_G):
        p = j % nbuf
        k = j + nbuf - 1
        if k < _G:
          q = k % nbuf
          if scat[q] is not None:
            scat[q].wait()
            scat[q] = None
          gath[q] = pltpu.async_copy(y.at[src_v.at[k]], rows[q], gsem[q])
        gath[p].wait()
        scat[p] = pltpu.make_async_copy(rows[p], acc.at[dst_v.at[j]],
                                        ssem[p])
        scat[p].start(add=True)

    def drain():
      for q in range(nbuf):
        if scat[q] is not None:
          scat[q].wait()
          scat[q] = None

    _scatter_pipeline(edges, wid, n_chunks,
                      [(src_v, 0), (dst_v, 1)], do_chunk, drain)
    plsc.subcore_barrier()

    wb = (n_nodes // _NS) // 8 * 8
    tail = n_nodes - _NS * wb
    pltpu.sync_copy(acc.at[pl.ds(sid * wb, wb)],
                    seg_out.at[cid, pl.ds(sid * wb, wb)])
    if tail:
      @pl.when(sid == 0)
      def _():
        pltpu.sync_copy(acc.at[pl.ds(_NS * wb, tail)],
                        seg_out.at[cid, pl.ds(_NS * wb, tail)])

  return pl.kernel(
      body,
      out_type=[jax.ShapeDtypeStruct((_NC, n_nodes, d_model), jnp.float32)],
      mesh=mesh, scratch_types=scratch, compiler_params=_SC_PARAMS)


def _cnt_sc(n_nodes, n_chunks):
  rpt = n_nodes // _NS
  mesh = plsc.VectorSubcoreMesh(core_axis_name="c", subcore_axis_name="s")

  scratch = [
      pltpu.VMEM_SHARED((n_nodes, 16), jnp.float32),
      pltpu.VMEM((_G, _C), jnp.int32),
      pltpu.VMEM((_C, 16), jnp.float32),
      pltpu.SemaphoreType.DMA,
      pltpu.SemaphoreType.DMA,
  ]

  def body(edges, ones, zcnt, cnt_out, cnt_sh, dst_v, ones_v, s0, s1):
    sems = (s0, s1)
    cid = lax.axis_index("c")
    sid = lax.axis_index("s")
    wid = sid * _NC + cid

    pltpu.sync_copy(ones, ones_v)
    pltpu.sync_copy(zcnt, cnt_sh.at[pl.ds(sid * rpt, rpt)])
    plsc.subcore_barrier()

    scat = [None, None]

    def do_chunk(g):
      del g
      for j in range(_G):
        p = j % 2
        if scat[p] is not None:
          scat[p].wait()
        scat[p] = pltpu.make_async_copy(ones_v, cnt_sh.at[dst_v.at[j]],
                                        sems[p])
        scat[p].start(add=True)

    def drain():
      for p in range(2):
        if scat[p] is not None:
          scat[p].wait()
          scat[p] = None

    _scatter_pipeline(edges, wid, n_chunks, [(dst_v, 1)], do_chunk, drain)
    plsc.subcore_barrier()

    wb = (n_nodes // _NS) // 8 * 8
    tail = n_nodes - _NS * wb
    pltpu.sync_copy(cnt_sh.at[pl.ds(sid * wb, wb)],
                    cnt_out.at[cid, pl.ds(sid * wb, wb)])
    if tail:
      @pl.when(sid == 0)
      def _():
        pltpu.sync_copy(cnt_sh.at[pl.ds(_NS * wb, tail)],
                        cnt_out.at[cid, pl.ds(_NS * wb, tail)])

  return pl.kernel(
      body,
      out_type=[jax.ShapeDtypeStruct((_NC, n_nodes, 16), jnp.float32)],
      mesh=mesh, scratch_types=scratch, compiler_params=_SC_PARAMS)


def _proj(x, W, b, block_n):
  n, d_in = x.shape
  d_out = W.shape[1]

  def body(x_ref, w_ref, *rest):
    out_ref = rest[-1]
    r = jnp.dot(x_ref[...], w_ref[...], preferred_element_type=jnp.float32)
    if b is not None:
      r = r + rest[0][...]
    out_ref[...] = r

  in_specs = [
      pl.BlockSpec((block_n, d_in), lambda i: (i, 0)),
      pl.BlockSpec((d_in, d_out), lambda i: (0, 0)),
  ]
  args = [x, W]
  if b is not None:
    in_specs.append(pl.BlockSpec((1, d_out), lambda i: (0, 0)))
    args.append(b.reshape(1, -1))
  return pl.pallas_call(
      body,
      grid=(n // block_n,),
      in_specs=in_specs,
      out_specs=pl.BlockSpec((block_n, d_out), lambda i: (i, 0)),
      out_shape=jax.ShapeDtypeStruct((n, d_out), jnp.float32),
  )(*args)


def _mid_layer(seg, cnt, z1, W1l, W2l, W2r, b2, bp):
  _, n, d_in = seg.shape
  d_h = W1l.shape[1]
  d_out = W2l.shape[1]
  half = n // 2
  nb = half // bp

  def body(s_t, c_t, z_t, s_b, c_b, z_b, w1, wl, wr, bb, y2p, z2p):
    def h_of(s_ref, c_ref, z_ref):
      c = jnp.maximum(c_ref[0, :, :1] + c_ref[1, :, :1], 1.0)
      mean = (s_ref[0] + s_ref[1]) / c
      return jnp.maximum(
          jnp.dot(mean, w1[...], preferred_element_type=jnp.float32)
          + z_ref[...], 0.0)

    ht = h_of(s_t, c_t, z_t)
    hb = h_of(s_b, c_b, z_b)
    y2p[...] = jnp.concatenate(
        [jnp.dot(ht, wl[...], preferred_element_type=jnp.float32),
         jnp.dot(hb, wl[...], preferred_element_type=jnp.float32)], axis=1)
    z2p[...] = jnp.concatenate(
        [jnp.dot(ht, wr[...], preferred_element_type=jnp.float32) + bb[...],
         jnp.dot(hb, wr[...], preferred_element_type=jnp.float32) + bb[...]],
        axis=1)

  return pl.pallas_call(
      body,
      grid=(nb,),
      in_specs=[
          pl.BlockSpec((2, bp, d_in), lambda i: (0, i, 0)),
          pl.BlockSpec((2, bp, 16), lambda i: (0, i, 0)),
          pl.BlockSpec((bp, d_h), lambda i: (i, 0)),
          pl.BlockSpec((2, bp, d_in), lambda i, _nb=nb: (0, i + _nb, 0)),
          pl.BlockSpec((2, bp, 16), lambda i, _nb=nb: (0, i + _nb, 0)),
          pl.BlockSpec((bp, d_h), lambda i, _nb=nb: (i + _nb, 0)),
          pl.BlockSpec((d_in, d_h), lambda i: (0, 0)),
          pl.BlockSpec((d_h, d_out), lambda i: (0, 0)),
          pl.BlockSpec((d_h, d_out), lambda i: (0, 0)),
          pl.BlockSpec((1, d_out), lambda i: (0, 0)),
      ],
      out_specs=[
          pl.BlockSpec((bp, 2 * d_out), lambda i: (i, 0)),
          pl.BlockSpec((bp, 2 * d_out), lambda i: (i, 0)),
      ],
      out_shape=[
          jax.ShapeDtypeStruct((half, 2 * d_out), jnp.float32),
          jax.ShapeDtypeStruct((half, 2 * d_out), jnp.float32),
      ],
  )(seg, cnt, z1, seg, cnt, z1, W1l, W2l, W2r, b2.reshape(1, -1))


def _final_layer(segp, cnt, z2p, bp):
  _, half, dd = segp.shape
  d_out = dd // 2
  nb = half // bp

  def body(s_ref, c_t, c_b, z_ref, out_t, out_b):
    def ls(v):
      m = jnp.max(v, axis=1, keepdims=True)
      e = jnp.exp(v - m)
      return v - m - jnp.log(jnp.sum(e, axis=1, keepdims=True))

    st = s_ref[0, :, :d_out] + s_ref[1, :, :d_out]
    sb = s_ref[0, :, d_out:] + s_ref[1, :, d_out:]
    ct = jnp.maximum(c_t[0, :, :1] + c_t[1, :, :1], 1.0)
    cb = jnp.maximum(c_b[0, :, :1] + c_b[1, :, :1], 1.0)
    out_t[...] = ls(st / ct + z_ref[:, :d_out])
    out_b[...] = ls(sb / cb + z_ref[:, d_out:])

  return pl.pallas_call(
      body,
      grid=(nb,),
      in_specs=[
          pl.BlockSpec((2, bp, dd), lambda i: (0, i, 0)),
          pl.BlockSpec((2, bp, 16), lambda i: (0, i, 0)),
          pl.BlockSpec((2, bp, 16), lambda i, _nb=nb: (0, i + _nb, 0)),
          pl.BlockSpec((bp, dd), lambda i: (i, 0)),
      ],
      out_specs=[
          pl.BlockSpec((bp, d_out), lambda i: (i, 0)),
          pl.BlockSpec((bp, d_out), lambda i: (i, 0)),
      ],
      out_shape=[
          jax.ShapeDtypeStruct((half, d_out), jnp.float32),
          jax.ShapeDtypeStruct((half, d_out), jnp.float32),
      ],
  )(segp, cnt, cnt, z2p)


def kernel(x, edge_index, W1l, W1r, b1, W2l, W2r, b2):
  n, d_in = x.shape
  e = edge_index.shape[1]
  d_h = W1l.shape[1]
  d_out = W2l.shape[1]

  n_chunks = e // (_NW * _C)
  block_n = 1000

  e4 = edge_index.reshape(2, _NW, n_chunks, _C)
  ones = jnp.ones((_C, 16), jnp.float32)
  zrow_h = jnp.zeros((n // _NS, d_in), jnp.float32)
  zrow_o = jnp.zeros((n // _NS, d_out), jnp.float32)
  zcnt = jnp.zeros((n // _NS, 16), jnp.float32)

  (cnt,) = _cnt_sc(n, n_chunks)(e4, ones, zcnt)

  (seg1,) = _seg_sum_sc(d_in, n, n_chunks, 3)(x, e4, zrow_h)
  z1 = _proj(x, W1r, b1, block_n)
  y2p, z2p = _mid_layer(seg1, cnt, z1, W1l, W2l, W2r, b2, block_n)

  e4p = (e4 % (n // 2)) * 2 + e4 // (n // 2)
  y2_lin = y2p.reshape(n, d_out)
  (seg2,) = _seg_sum_sc(d_out, n, n_chunks, 6)(y2_lin, e4p, zrow_o)
  segp = seg2.reshape(2, n // 2, 2 * d_out)
  out_t, out_b = _final_layer(segp, cnt, z2p, block_n)
  return jnp.concatenate([out_t, out_b], axis=0)


# speedup vs baseline: 3.1427x; 1.0000x over previous
import jax
import jax.numpy as jnp
from jax import lax
from jax.experimental import pallas as pl
from jax.experimental.pallas import tpu as pltpu
from jax.experimental.pallas import tpu_sc as plsc

_NC = 2
_NS = 16
_NW = _NC * _NS

_C = 100
_G = 20

_SC_PARAMS = pltpu.CompilerParams(use_tc_tiling_on_sc=False)


def _scatter_pipeline(edges, wid, n_chunks, idx_bufs, do_chunk, drain):
  n_groups = n_chunks // _G

  def group(g, carry):
    for ref, plane in idx_bufs:
      pltpu.sync_copy(edges.at[plane, wid, pl.ds(g * _G, _G)], ref)
    do_chunk(g)
    drain()
    return carry

  lax.fori_loop(0, n_groups, group, 0)


def _seg_sum_sc(d_model, n_nodes, n_chunks, nbuf):
  rpt = n_nodes // _NS
  mesh = plsc.VectorSubcoreMesh(core_axis_name="c", subcore_axis_name="s")

  scratch = [
      pltpu.VMEM_SHARED((n_nodes, d_model), jnp.float32),
      pltpu.VMEM((_G, _C), jnp.int32),
      pltpu.VMEM((_G, _C), jnp.int32),
  ]
  scratch += [pltpu.VMEM((_C, d_model), jnp.float32) for _ in range(nbuf)]
  scratch += [pltpu.SemaphoreType.DMA for _ in range(2 * nbuf)]

  def body(y, edges, zrow, seg_out, acc, src_v, dst_v, *bufs):
    rows = bufs[:nbuf]
    gsem = bufs[nbuf:2 * nbuf]
    ssem = bufs[2 * nbuf:]
    cid = lax.axis_index("c")
    sid = lax.axis_index("s")
    wid = sid * _NC + cid

    pltpu.sync_copy(zrow, acc.at[pl.ds(sid * rpt, rpt)])
    plsc.subcore_barrier()

    gath = [None] * nbuf
    scat = [None] * nbuf

    def do_chunk(g):
      del g
      for q in range(min(nbuf - 1, _G)):
        gath[q] = pltpu.async_copy(y.at[src_v.at[q]], rows[q], gsem[q])
      for j in range(_G):
        p = j % nbuf
        k = j + nbuf - 1
        if k < _G:
          q = k % nbuf
          if scat[q] is not None:
            scat[q].wait()
            scat[q] = None
          gath[q] = pltpu.async_copy(y.at[src_v.at[k]], rows[q], gsem[q])
        gath[p].wait()
        scat[p] = pltpu.make_async_copy(rows[p], acc.at[dst_v.at[j]],
                                        ssem[p])
        scat[p].start(add=True)

    def drain():
      for q in range(nbuf):
        if scat[q] is not None:
          scat[q].wait()
          scat[q] = None

    _scatter_pipeline(edges, wid, n_chunks,
                      [(src_v, 0), (dst_v, 1)], do_chunk, drain)
    plsc.subcore_barrier()

    wb = (n_nodes // _NS) // 8 * 8
    tail = n_nodes - _NS * wb
    pltpu.sync_copy(acc.at[pl.ds(sid * wb, wb)],
                    seg_out.at[cid, pl.ds(sid * wb, wb)])
    if tail:
      @pl.when(sid == 0)
      def _():
        pltpu.sync_copy(acc.at[pl.ds(_NS * wb, tail)],
                        seg_out.at[cid, pl.ds(_NS * wb, tail)])

  return pl.kernel(
      body,
      out_type=[jax.ShapeDtypeStruct((_NC, n_nodes, d_model), jnp.float32)],
      mesh=mesh, scratch_types=scratch, compiler_params=_SC_PARAMS)


def _cnt_sc(n_nodes, n_chunks):
  rpt = n_nodes // _NS
  mesh = plsc.VectorSubcoreMesh(core_axis_name="c", subcore_axis_name="s")

  scratch = [
      pltpu.VMEM_SHARED((n_nodes, 16), jnp.float32),
      pltpu.VMEM((_G, _C), jnp.int32),
      pltpu.VMEM((_C, 16), jnp.float32),
      pltpu.SemaphoreType.DMA,
      pltpu.SemaphoreType.DMA,
  ]

  def body(edges, ones, zcnt, cnt_out, cnt_sh, dst_v, ones_v, s0, s1):
    sems = (s0, s1)
    cid = lax.axis_index("c")
    sid = lax.axis_index("s")
    wid = sid * _NC + cid

    pltpu.sync_copy(ones, ones_v)
    pltpu.sync_copy(zcnt, cnt_sh.at[pl.ds(sid * rpt, rpt)])
    plsc.subcore_barrier()

    scat = [None, None]

    def do_chunk(g):
      del g
      for j in range(_G):
        p = j % 2
        if scat[p] is not None:
          scat[p].wait()
        scat[p] = pltpu.make_async_copy(ones_v, cnt_sh.at[dst_v.at[j]],
                                        sems[p])
        scat[p].start(add=True)

    def drain():
      for p in range(2):
        if scat[p] is not None:
          scat[p].wait()
          scat[p] = None

    _scatter_pipeline(edges, wid, n_chunks, [(dst_v, 1)], do_chunk, drain)
    plsc.subcore_barrier()

    wb = (n_nodes // _NS) // 8 * 8
    tail = n_nodes - _NS * wb
    pltpu.sync_copy(cnt_sh.at[pl.ds(sid * wb, wb)],
                    cnt_out.at[cid, pl.ds(sid * wb, wb)])
    if tail:
      @pl.when(sid == 0)
      def _():
        pltpu.sync_copy(cnt_sh.at[pl.ds(_NS * wb, tail)],
                        cnt_out.at[cid, pl.ds(_NS * wb, tail)])

  return pl.kernel(
      body,
      out_type=[jax.ShapeDtypeStruct((_NC, n_nodes, 16), jnp.float32)],
      mesh=mesh, scratch_types=scratch, compiler_params=_SC_PARAMS)


def _proj(x, W, b, block_n):
  n, d_in = x.shape
  d_out = W.shape[1]

  def body(x_ref, w_ref, *rest):
    out_ref = rest[-1]
    r = jnp.dot(x_ref[...], w_ref[...], preferred_element_type=jnp.float32)
    if b is not None:
      r = r + rest[0][...]
    out_ref[...] = r

  in_specs = [
      pl.BlockSpec((block_n, d_in), lambda i: (i, 0)),
      pl.BlockSpec((d_in, d_out), lambda i: (0, 0)),
  ]
  args = [x, W]
  if b is not None:
    in_specs.append(pl.BlockSpec((1, d_out), lambda i: (0, 0)))
    args.append(b.reshape(1, -1))
  return pl.pallas_call(
      body,
      grid=(n // block_n,),
      in_specs=in_specs,
      out_specs=pl.BlockSpec((block_n, d_out), lambda i: (i, 0)),
      out_shape=jax.ShapeDtypeStruct((n, d_out), jnp.float32),
  )(*args)


def _mid_layer(seg, cnt, z1, W1l, W2l, W2r, b2, bp):
  _, n, d_in = seg.shape
  d_h = W1l.shape[1]
  d_out = W2l.shape[1]
  half = n // 2
  nb = half // bp

  def body(s_t, c_t, z_t, s_b, c_b, z_b, w1, wl, wr, bb, y2p, z2p):
    def h_of(s_ref, c_ref, z_ref):
      c = jnp.maximum(c_ref[0, :, :1] + c_ref[1, :, :1], 1.0)
      mean = (s_ref[0] + s_ref[1]) / c
      return jnp.maximum(
          jnp.dot(mean, w1[...], preferred_element_type=jnp.float32)
          + z_ref[...], 0.0)

    ht = h_of(s_t, c_t, z_t)
    hb = h_of(s_b, c_b, z_b)
    y2p[...] = jnp.concatenate(
        [jnp.dot(ht, wl[...], preferred_element_type=jnp.float32),
         jnp.dot(hb, wl[...], preferred_element_type=jnp.float32)], axis=1)
    z2p[...] = jnp.concatenate(
        [jnp.dot(ht, wr[...], preferred_element_type=jnp.float32) + bb[...],
         jnp.dot(hb, wr[...], preferred_element_type=jnp.float32) + bb[...]],
        axis=1)

  return pl.pallas_call(
      body,
      grid=(nb,),
      in_specs=[
          pl.BlockSpec((2, bp, d_in), lambda i: (0, i, 0)),
          pl.BlockSpec((2, bp, 16), lambda i: (0, i, 0)),
          pl.BlockSpec((bp, d_h), lambda i: (i, 0)),
          pl.BlockSpec((2, bp, d_in), lambda i, _nb=nb: (0, i + _nb, 0)),
          pl.BlockSpec((2, bp, 16), lambda i, _nb=nb: (0, i + _nb, 0)),
          pl.BlockSpec((bp, d_h), lambda i, _nb=nb: (i + _nb, 0)),
          pl.BlockSpec((d_in, d_h), lambda i: (0, 0)),
          pl.BlockSpec((d_h, d_out), lambda i: (0, 0)),
          pl.BlockSpec((d_h, d_out), lambda i: (0, 0)),
          pl.BlockSpec((1, d_out), lambda i: (0, 0)),
      ],
      out_specs=[
          pl.BlockSpec((bp, 2 * d_out), lambda i: (i, 0)),
          pl.BlockSpec((bp, 2 * d_out), lambda i: (i, 0)),
      ],
      out_shape=[
          jax.ShapeDtypeStruct((half, 2 * d_out), jnp.float32),
          jax.ShapeDtypeStruct((half, 2 * d_out), jnp.float32),
      ],
  )(seg, cnt, z1, seg, cnt, z1, W1l, W2l, W2r, b2.reshape(1, -1))


def _final_layer(segp, cnt, z2p, bp):
  _, half, dd = segp.shape
  d_out = dd // 2
  nb = half // bp

  def body(s_ref, c_t, c_b, z_ref, out_t, out_b):
    def ls(v):
      m = jnp.max(v, axis=1, keepdims=True)
      e = jnp.exp(v - m)
      return v - m - jnp.log(jnp.sum(e, axis=1, keepdims=True))

    st = s_ref[0, :, :d_out] + s_ref[1, :, :d_out]
    sb = s_ref[0, :, d_out:] + s_ref[1, :, d_out:]
    ct = jnp.maximum(c_t[0, :, :1] + c_t[1, :, :1], 1.0)
    cb = jnp.maximum(c_b[0, :, :1] + c_b[1, :, :1], 1.0)
    out_t[...] = ls(st / ct + z_ref[:, :d_out])
    out_b[...] = ls(sb / cb + z_ref[:, d_out:])

  return pl.pallas_call(
      body,
      grid=(nb,),
      in_specs=[
          pl.BlockSpec((2, bp, dd), lambda i: (0, i, 0)),
          pl.BlockSpec((2, bp, 16), lambda i: (0, i, 0)),
          pl.BlockSpec((2, bp, 16), lambda i, _nb=nb: (0, i + _nb, 0)),
          pl.BlockSpec((bp, dd), lambda i: (i, 0)),
      ],
      out_specs=[
          pl.BlockSpec((bp, d_out), lambda i: (i, 0)),
          pl.BlockSpec((bp, d_out), lambda i: (i, 0)),
      ],
      out_shape=[
          jax.ShapeDtypeStruct((half, d_out), jnp.float32),
          jax.ShapeDtypeStruct((half, d_out), jnp.float32),
      ],
  )(segp, cnt, cnt, z2p)


def kernel(x, edge_index, W1l, W1r, b1, W2l, W2r, b2):
  n, d_in = x.shape
  e = edge_index.shape[1]
  d_h = W1l.shape[1]
  d_out = W2l.shape[1]

  n_chunks = e // (_NW * _C)
  block_n = 1000

  e4 = edge_index.reshape(2, _NW, n_chunks, _C)
  ones = jnp.ones((_C, 16), jnp.float32)
  zrow_h = jnp.zeros((n // _NS, d_in), jnp.float32)
  zrow_o = jnp.zeros((n // _NS, d_out), jnp.float32)
  zcnt = jnp.zeros((n // _NS, 16), jnp.float32)

  (cnt,) = _cnt_sc(n, n_chunks)(e4, ones, zcnt)

  (seg1,) = _seg_sum_sc(d_in, n, n_chunks, 3)(x, e4, zrow_h)
  z1 = _proj(x, W1r, b1, block_n)
  y2p, z2p = _mid_layer(seg1, cnt, z1, W1l, W2l, W2r, b2, block_n)

  e4b, z1 = lax.optimization_barrier((e4, z1))
  e4p = (e4b % (n // 2)) * 2 + e4b // (n // 2)
  y2_lin = y2p.reshape(n, d_out)
  (seg2,) = _seg_sum_sc(d_out, n, n_chunks, 6)(y2_lin, e4p, zrow_o)
  segp = seg2.reshape(2, n // 2, 2 * d_out)
  out_t, out_b = _final_layer(segp, cnt, z2p, block_n)
  return jnp.concatenate([out_t, out_b], axis=0)
